# baseline (speedup 1.0000x reference)
import functools

import jax
import jax.numpy as jnp
from jax import lax
from jax.experimental import pallas as pl
from jax.experimental.pallas import tpu as pltpu
from jax.experimental.pallas import tpu_sc as plsc

_TILE_V = 5120


def _make_sc_gather(B, L, V, H):
    info = plsc.get_sparse_core_info()
    NC, NS = info.num_cores, info.num_subcores
    NW = NC * NS
    b_per_w = B * L // NW
    mesh = plsc.VectorSubcoreMesh(core_axis_name="c", subcore_axis_name="s")
    two_d = B == NW and b_per_w == L

    @functools.partial(
        pl.kernel,
        mesh=mesh,
        out_type=jax.ShapeDtypeStruct((B * L, H), jnp.float32),
        scratch_types=[
            pltpu.VMEM((b_per_w,), jnp.int32),
            pltpu.VMEM((b_per_w, H), jnp.float32),
            pltpu.SemaphoreType.DMA,
        ],
    )
    def gather_k(idx_hbm, table_hbm, out_hbm, idx_v, rows_v, sem):
        wid = lax.axis_index("s") * NC + lax.axis_index("c")
        base = wid * b_per_w
        if two_d:
            pltpu.sync_copy(idx_hbm.at[wid], idx_v)
        else:
            pltpu.sync_copy(idx_hbm.at[pl.ds(base, b_per_w)], idx_v)
        pltpu.async_copy(table_hbm.at[idx_v], rows_v, sem).wait()
        pltpu.sync_copy(rows_v, out_hbm.at[pl.ds(base, b_per_w)])

    return gather_k, two_d


def _matmul_bias(x, w, b, tile_v):
    Bt, H = x.shape
    V = w.shape[0]
    nv = pl.cdiv(V, tile_v)

    def body(x_ref, w_ref, b_ref, o_ref):
        o_ref[...] = (
            lax.dot_general(
                x_ref[...],
                w_ref[...],
                (((1,), (1,)), ((), ())),
                preferred_element_type=jnp.float32,
            )
            + b_ref[...][None, :]
        )

    return pl.pallas_call(
        body,
        grid=(nv,),
        in_specs=[
            pl.BlockSpec((Bt, H), lambda i: (0, 0)),
            pl.BlockSpec((tile_v, H), lambda i: (i, 0)),
            pl.BlockSpec((tile_v,), lambda i: (i,)),
        ],
        out_specs=pl.BlockSpec((Bt, tile_v), lambda i: (0, i)),
        out_shape=jax.ShapeDtypeStruct((Bt, V), jnp.float32),
    )(x, w, b)


def kernel(input_ids, embedding, lm_head_w, lm_head_b):
    B, L = input_ids.shape
    V, H = embedding.shape
    ids = input_ids if input_ids.dtype == jnp.int32 else input_ids.astype(jnp.int32)
    gather_k, two_d = _make_sc_gather(B, L, V, H)
    embeds = gather_k(ids if two_d else ids.reshape(B * L), embedding)
    logits = _matmul_bias(embeds, lm_head_w, lm_head_b, _TILE_V)
    return logits.reshape(B, L, V)
